# baseline (speedup 1.0000x reference)
import functools

import jax
import jax.numpy as jnp
from jax import lax
from jax.experimental import pallas as pl
from jax.experimental.pallas import tpu as pltpu
from jax.experimental.pallas import tpu_sc as plsc

G = 128
TILES = 16


def _mlp(y, w1, b1, w2, b2):
    h = jnp.maximum(jnp.dot(y, w1, preferred_element_type=jnp.float32) + b1, 0.0)
    return jnp.dot(h, w2, preferred_element_type=jnp.float32) + b2


def _mlp0_body(wf, W1, b1, W2, b2, out_ref):
    out_ref[...] = _mlp(wf[...], W1[...], b1[...], W2[...], b2[...])


def _run_mlp0(walk_feats, W0_1, b0_1r, W0_2, b0_2r):
    n = walk_feats.shape[0]
    h2dim = W0_2.shape[1]
    return pl.pallas_call(
        _mlp0_body,
        out_shape=jax.ShapeDtypeStruct((n, h2dim), jnp.float32),
    )(walk_feats, W0_1, b0_1r, W0_2, b0_2r)


def _sc_pool_body(out0_hbm, seg_hbm, zeros_hbm, iota_hbm, out_hbm,
                  ids_v, rows_v, iota_v, comb_v, shared):
    cid = lax.axis_index("c")
    sid = lax.axis_index("s")

    @pl.when(cid == 0)
    def _work():
        pltpu.sync_copy(zeros_hbm, shared.at[pl.ds(sid * G, G)])
        pltpu.sync_copy(seg_hbm.at[sid], ids_v)
        pltpu.sync_copy(iota_hbm.at[sid], iota_v)
        pltpu.sync_copy(out0_hbm.at[sid], rows_v)
        plsc.subcore_barrier()
        nchunks = ids_v.shape[0]
        for j in range(nchunks):
            pltpu.sync_copy(rows_v.at[pl.ds(j * 128, 128)],
                            shared.at[ids_v.at[j]], add=True)
        plsc.subcore_barrier()

        @pl.when(sid == 0)
        def _combine():
            for t in range(1, TILES):
                pltpu.sync_copy(shared.at[pl.ds(t * G, G)], comb_v)
                pltpu.sync_copy(comb_v, shared.at[iota_v.at[0]], add=True)
            pltpu.sync_copy(shared.at[pl.ds(0, G)], out_hbm)


def _run_sc_pool(out0, segment_ids):
    n, hdim = out0.shape
    rows_per_tile = -(-n // (TILES * 128)) * 128
    npad = rows_per_tile * TILES
    nchunks = rows_per_tile // 128
    seg2 = (jnp.pad(segment_ids.astype(jnp.int32), (0, npad - n))
            .reshape(TILES, nchunks, 128)
            + (jnp.arange(TILES, dtype=jnp.int32) * G)[:, None, None])
    out0r = jnp.pad(out0, ((0, npad - n), (0, 0))).reshape(
        TILES, rows_per_tile, hdim)
    zeros = jnp.zeros((G, hdim), jnp.float32)
    iota = (jnp.arange(TILES, dtype=jnp.int32)[:, None] * G
            + jnp.arange(G, dtype=jnp.int32)[None, :]).reshape(TILES, 1, G)

    mesh = plsc.VectorSubcoreMesh(core_axis_name="c", subcore_axis_name="s")
    fn = functools.partial(
        pl.kernel,
        mesh=mesh,
        out_type=jax.ShapeDtypeStruct((G, hdim), jnp.float32),
        scratch_types=[
            pltpu.VMEM((nchunks, 128), jnp.int32),
            pltpu.VMEM((rows_per_tile, hdim), jnp.float32),
            pltpu.VMEM((1, G), jnp.int32),
            pltpu.VMEM((G, hdim), jnp.float32),
            pltpu.VMEM_SHARED((TILES * G, hdim), jnp.float32),
        ],
    )(_sc_pool_body)
    return fn(out0r, seg2, zeros, iota)


def _hops_body(x_full, h1, h2, h3, seg,
               W1_1, b1_1, W1_2, b1_2,
               W2_1, b2_1, W2_2, b2_2,
               W3_1, b3_1, W3_2, b3_2,
               out_ref, *, block_rows):
    i = pl.program_id(0)
    X = x_full[...]

    y = _mlp(jnp.dot(h1[...], X, preferred_element_type=jnp.float32),
             W1_1[...], b1_1[...], W1_2[...], b1_2[...])
    y = y + _mlp(jnp.dot(h2[...], X, preferred_element_type=jnp.float32),
                 W2_1[...], b2_1[...], W2_2[...], b2_2[...])
    y = y + _mlp(jnp.dot(h3[...], X, preferred_element_type=jnp.float32),
                 W3_1[...], b3_1[...], W3_2[...], b3_2[...])

    ids = jnp.broadcast_to(seg[0], (G, block_rows))
    onehotT = (lax.broadcasted_iota(jnp.int32, (G, block_rows), 0) == ids)
    contrib = jnp.dot(onehotT.astype(jnp.float32), y,
                      preferred_element_type=jnp.float32)

    @pl.when(i == 0)
    def _init():
        out_ref[...] = contrib

    @pl.when(i > 0)
    def _acc():
        out_ref[...] += contrib


def _run_hops(walk_feats, hop1, hop2, hop3, segment_ids, weights):
    n, rw = walk_feats.shape
    (W1_1, b1_1r, W1_2, b1_2r,
     W2_1, b2_1r, W2_2, b2_2r,
     W3_1, b3_1r, W3_2, b3_2r) = weights
    h2dim = W1_2.shape[1]
    block_rows = 200
    nblk = n // block_rows
    seg3 = segment_ids.astype(jnp.int32).reshape(nblk, 1, block_rows)

    hop_spec = pl.BlockSpec((block_rows, n), lambda i: (i, 0))
    full = lambda a: pl.BlockSpec(a.shape, lambda i: (0,) * a.ndim)

    grid_spec = pl.GridSpec(
        grid=(nblk,),
        in_specs=[
            full(walk_feats),
            hop_spec, hop_spec, hop_spec,
            pl.BlockSpec((1, 1, block_rows), lambda i: (i, 0, 0)),
            full(W1_1), full(b1_1r), full(W1_2), full(b1_2r),
            full(W2_1), full(b2_1r), full(W2_2), full(b2_2r),
            full(W3_1), full(b3_1r), full(W3_2), full(b3_2r),
        ],
        out_specs=pl.BlockSpec((G, h2dim), lambda i: (0, 0)),
    )

    return pl.pallas_call(
        functools.partial(_hops_body, block_rows=block_rows),
        grid_spec=grid_spec,
        out_shape=jax.ShapeDtypeStruct((G, h2dim), jnp.float32),
        compiler_params=pltpu.CompilerParams(
            dimension_semantics=("arbitrary",),
        ),
    )(walk_feats, hop1, hop2, hop3, seg3,
      W1_1, b1_1r, W1_2, b1_2r,
      W2_1, b2_1r, W2_2, b2_2r,
      W3_1, b3_1r, W3_2, b3_2r)


def kernel(x, walk_feats, hop1, hop2, hop3, segment_ids,
           W0_1, b0_1, W0_2, b0_2,
           W1_1, b1_1, W1_2, b1_2,
           W2_1, b2_1, W2_2, b2_2,
           W3_1, b3_1, W3_2, b3_2):
    del x
    biases = [b.reshape(1, -1) for b in (b0_1, b0_2, b1_1, b1_2,
                                         b2_1, b2_2, b3_1, b3_2)]
    (b0_1r, b0_2r, b1_1r, b1_2r, b2_1r, b2_2r, b3_1r, b3_2r) = biases

    out0 = _run_mlp0(walk_feats, W0_1, b0_1r, W0_2, b0_2r)
    pooled0 = _run_sc_pool(out0, segment_ids)
    pooled123 = _run_hops(
        walk_feats, hop1, hop2, hop3, segment_ids,
        (W1_1, b1_1r, W1_2, b1_2r,
         W2_1, b2_1r, W2_2, b2_2r,
         W3_1, b3_1r, W3_2, b3_2r))
    return pooled123 + pooled0


# speedup vs baseline: 1.0703x; 1.0703x over previous
import functools

import jax
import jax.numpy as jnp
from jax import lax
from jax.experimental import pallas as pl
from jax.experimental.pallas import tpu as pltpu

G = 128


def _mlp(y, w1, b1, w2, b2):
    h = jnp.maximum(jnp.dot(y, w1, preferred_element_type=jnp.float32) + b1, 0.0)
    return jnp.dot(h, w2, preferred_element_type=jnp.float32) + b2


def _body(wf_blk, x_full, h1, h2, h3, seg,
          W0_1, b0_1, W0_2, b0_2,
          W1_1, b1_1, W1_2, b1_2,
          W2_1, b2_1, W2_2, b2_2,
          W3_1, b3_1, W3_2, b3_2,
          out_ref, *, block_rows):
    i = pl.program_id(0)
    X = x_full[...]

    y = _mlp(wf_blk[...], W0_1[...], b0_1[...], W0_2[...], b0_2[...])
    y = y + _mlp(jnp.dot(h1[...], X, preferred_element_type=jnp.float32),
                 W1_1[...], b1_1[...], W1_2[...], b1_2[...])
    y = y + _mlp(jnp.dot(h2[...], X, preferred_element_type=jnp.float32),
                 W2_1[...], b2_1[...], W2_2[...], b2_2[...])
    y = y + _mlp(jnp.dot(h3[...], X, preferred_element_type=jnp.float32),
                 W3_1[...], b3_1[...], W3_2[...], b3_2[...])

    ids = jnp.broadcast_to(seg[0], (G, block_rows))
    onehotT = (lax.broadcasted_iota(jnp.int32, (G, block_rows), 0) == ids)
    contrib = jnp.dot(onehotT.astype(jnp.float32), y,
                      preferred_element_type=jnp.float32)

    @pl.when(i == 0)
    def _init():
        out_ref[...] = contrib

    @pl.when(i > 0)
    def _acc():
        out_ref[...] += contrib


def kernel(x, walk_feats, hop1, hop2, hop3, segment_ids,
           W0_1, b0_1, W0_2, b0_2,
           W1_1, b1_1, W1_2, b1_2,
           W2_1, b2_1, W2_2, b2_2,
           W3_1, b3_1, W3_2, b3_2):
    del x
    n, rw = walk_feats.shape
    h2dim = W0_2.shape[1]
    block_rows = 200
    assert n % block_rows == 0
    nblk = n // block_rows

    seg3 = segment_ids.astype(jnp.int32).reshape(nblk, 1, block_rows)
    biases = [b.reshape(1, -1) for b in (b0_1, b0_2, b1_1, b1_2,
                                         b2_1, b2_2, b3_1, b3_2)]
    (b0_1r, b0_2r, b1_1r, b1_2r, b2_1r, b2_2r, b3_1r, b3_2r) = biases

    row_spec = pl.BlockSpec((block_rows, rw), lambda i: (i, 0))
    hop_spec = pl.BlockSpec((block_rows, n), lambda i: (i, 0))
    full = lambda a: pl.BlockSpec(a.shape, lambda i: (0,) * a.ndim)

    grid_spec = pl.GridSpec(
        grid=(nblk,),
        in_specs=[
            row_spec,
            full(walk_feats),
            hop_spec, hop_spec, hop_spec,
            pl.BlockSpec((1, 1, block_rows), lambda i: (i, 0, 0)),
            full(W0_1), full(b0_1r), full(W0_2), full(b0_2r),
            full(W1_1), full(b1_1r), full(W1_2), full(b1_2r),
            full(W2_1), full(b2_1r), full(W2_2), full(b2_2r),
            full(W3_1), full(b3_1r), full(W3_2), full(b3_2r),
        ],
        out_specs=pl.BlockSpec((G, h2dim), lambda i: (0, 0)),
    )

    return pl.pallas_call(
        functools.partial(_body, block_rows=block_rows),
        grid_spec=grid_spec,
        out_shape=jax.ShapeDtypeStruct((G, h2dim), jnp.float32),
        compiler_params=pltpu.CompilerParams(
            dimension_semantics=("arbitrary",),
        ),
    )(walk_feats, walk_feats, hop1, hop2, hop3, seg3,
      W0_1, b0_1r, W0_2, b0_2r,
      W1_1, b1_1r, W1_2, b1_2r,
      W2_1, b2_1r, W2_2, b2_2r,
      W3_1, b3_1r, W3_2, b3_2r)
